# Initial kernel scaffold; baseline (speedup 1.0000x reference)
import functools

import jax
import jax.numpy as jnp
from jax import lax
from jax.experimental import pallas as pl
from jax.experimental.pallas import tpu as pltpu
from jax.experimental.pallas import tpu_sc as plsc

N = 10000
E = 320000
D_IN = 128
D_H = 256
D_OUT = 40

NC = 2
NS = 16
NT = NC * NS
BATCH = 128
NB = -(-E // (NT * BATCH))
EP = NB * BATCH * NT
NP = 10240
RT = NP // NS
BN = 256
GRID = NP // BN


def _sc_mesh():
  return plsc.VectorSubcoreMesh(core_axis_name="c", subcore_axis_name="s")


@functools.partial(
    pl.kernel,
    out_type=jax.ShapeDtypeStruct((NC * NP,), jnp.float32),
    mesh=_sc_mesh(),
    scratch_types=[
        pltpu.VMEM((NB, BATCH), jnp.int32),
        pltpu.VMEM((BATCH,), jnp.float32),
        pltpu.VMEM((RT,), jnp.float32),
        pltpu.VMEM_SHARED((NP,), jnp.float32),
    ],
)
def _deg_kernel(dstr_hbm, ones_hbm, zeros_hbm, out_hbm, dst_v, ones_v, zrow_v,
                acc):
  c = lax.axis_index("c")
  s = lax.axis_index("s")
  w = s * NC + c
  pltpu.sync_copy(zeros_hbm, zrow_v)
  pltpu.sync_copy(zrow_v, acc.at[pl.ds(s * RT, RT)])
  pltpu.sync_copy(ones_hbm, ones_v)
  pltpu.sync_copy(dstr_hbm.at[pl.ds(w * NB, NB)], dst_v)
  plsc.subcore_barrier()

  def body(j, carry):
    pltpu.sync_copy(ones_v, acc.at[dst_v.at[j]], add=True)
    return carry

  lax.fori_loop(0, NB, body, 0)
  plsc.subcore_barrier()
  pltpu.sync_copy(acc.at[pl.ds(s * RT, RT)],
                  out_hbm.at[pl.ds(c * NP + s * RT, RT)])


def _make_agg_kernel(D):
  @functools.partial(
      pl.kernel,
      out_type=jax.ShapeDtypeStruct((NC * NP, D), jnp.float32),
      mesh=_sc_mesh(),
      scratch_types=[
          pltpu.VMEM((NB, BATCH), jnp.int32),
          pltpu.VMEM((NB, BATCH), jnp.int32),
          pltpu.VMEM((BATCH, D), jnp.float32),
          pltpu.VMEM_SHARED((NP, D), jnp.float32),
          pltpu.SemaphoreType.DMA,
      ],
  )
  def agg(xs_hbm, srcr_hbm, dstr_hbm, zeros_hbm, out_hbm, src_v, dst_v,
          rows_a, acc, sem_a):
    c = lax.axis_index("c")
    s = lax.axis_index("s")
    w = s * NC + c
    pltpu.sync_copy(zeros_hbm, rows_a)
    for k in range(RT // BATCH):
      pltpu.sync_copy(rows_a, acc.at[pl.ds(s * RT + k * BATCH, BATCH)])
    pltpu.sync_copy(srcr_hbm.at[pl.ds(w * NB, NB)], src_v)
    pltpu.sync_copy(dstr_hbm.at[pl.ds(w * NB, NB)], dst_v)
    plsc.subcore_barrier()

    def body(j, carry):
      pltpu.async_copy(xs_hbm.at[src_v.at[j]], rows_a, sem_a).wait()
      pltpu.sync_copy(rows_a, acc.at[dst_v.at[j]], add=True)
      return carry

    lax.fori_loop(0, NB, body, 0)
    plsc.subcore_barrier()
    pltpu.sync_copy(acc.at[pl.ds(s * RT, RT)],
                    out_hbm.at[pl.ds(c * NP + s * RT, RT)])

  return agg


_agg128 = _make_agg_kernel(D_IN)
_agg40 = _make_agg_kernel(D_OUT)


def _scale_body(degp_ref, x_ref, dinv_ref, xs_ref):
  deg = degp_ref[0] + degp_ref[1] + 1.0
  dinv = lax.rsqrt(deg)
  dinv_ref[...] = dinv
  xs_ref[...] = x_ref[...] * dinv


def _scale_call(degp, x_p):
  return pl.pallas_call(
      _scale_body,
      grid=(GRID,),
      in_specs=[
          pl.BlockSpec((NC, BN, 1), lambda i: (0, i, 0)),
          pl.BlockSpec((BN, D_IN), lambda i: (i, 0)),
      ],
      out_specs=[
          pl.BlockSpec((BN, 1), lambda i: (i, 0)),
          pl.BlockSpec((BN, D_IN), lambda i: (i, 0)),
      ],
      out_shape=[
          jax.ShapeDtypeStruct((NP, 1), jnp.float32),
          jax.ShapeDtypeStruct((NP, D_IN), jnp.float32),
      ],
  )(degp, x_p)


def _mlp_body(p1_ref, xs_ref, dinv_ref, w1_ref, b1_ref, w2_ref, zs_ref):
  i = pl.program_id(0)
  dinv = dinv_ref[...]
  agg = (p1_ref[0] + p1_ref[1] + xs_ref[...]) * dinv
  h = jnp.maximum(
      jnp.dot(agg, w1_ref[...], preferred_element_type=jnp.float32)
      + b1_ref[...], 0.0)
  z = jnp.dot(h, w2_ref[...], preferred_element_type=jnp.float32)
  row = i * BN + lax.broadcasted_iota(jnp.int32, (BN, 1), 0)
  zs_ref[...] = jnp.where(row < N, z * dinv, 0.0)


def _mlp_call(p1, xs, dinv, W1, b1, W2):
  return pl.pallas_call(
      _mlp_body,
      grid=(GRID,),
      in_specs=[
          pl.BlockSpec((NC, BN, D_IN), lambda i: (0, i, 0)),
          pl.BlockSpec((BN, D_IN), lambda i: (i, 0)),
          pl.BlockSpec((BN, 1), lambda i: (i, 0)),
          pl.BlockSpec((D_IN, D_H), lambda i: (0, 0)),
          pl.BlockSpec((1, D_H), lambda i: (0, 0)),
          pl.BlockSpec((D_H, D_OUT), lambda i: (0, 0)),
      ],
      out_specs=pl.BlockSpec((BN, D_OUT), lambda i: (i, 0)),
      out_shape=jax.ShapeDtypeStruct((NP, D_OUT), jnp.float32),
  )(p1, xs, dinv, W1, b1, W2)


def _loss_body(p2_ref, zs_ref, dinv_ref, b2_ref, y_ref, logits_ref, loss_ref):
  i = pl.program_id(0)
  logits = (p2_ref[0] + p2_ref[1] + zs_ref[...]) * dinv_ref[...] + b2_ref[...]
  logits_ref[...] = logits
  m = jnp.max(logits, axis=1, keepdims=True)
  lse = jnp.log(jnp.sum(jnp.exp(logits - m), axis=1, keepdims=True)) + m
  sel = lax.broadcasted_iota(jnp.int32, (BN, D_OUT), 1) == y_ref[...]
  picked = jnp.sum(jnp.where(sel, logits, 0.0), axis=1, keepdims=True)
  row = i * BN + lax.broadcasted_iota(jnp.int32, (BN, 1), 0)
  part = jnp.sum(jnp.where(row < N, lse - picked, 0.0))

  @pl.when(i == 0)
  def _():
    loss_ref[0, 0] = 0.0

  loss_ref[0, 0] += part


def _loss_call(p2, zs, dinv, b2, y_p):
  return pl.pallas_call(
      _loss_body,
      grid=(GRID,),
      in_specs=[
          pl.BlockSpec((NC, BN, D_OUT), lambda i: (0, i, 0)),
          pl.BlockSpec((BN, D_OUT), lambda i: (i, 0)),
          pl.BlockSpec((BN, 1), lambda i: (i, 0)),
          pl.BlockSpec((1, D_OUT), lambda i: (0, 0)),
          pl.BlockSpec((BN, 1), lambda i: (i, 0)),
      ],
      out_specs=[
          pl.BlockSpec((BN, D_OUT), lambda i: (i, 0)),
          pl.BlockSpec((1, 1), lambda i: (0, 0)),
      ],
      out_shape=[
          jax.ShapeDtypeStruct((NP, D_OUT), jnp.float32),
          jax.ShapeDtypeStruct((1, 1), jnp.float32),
      ],
  )(p2, zs, dinv, b2, y_p)


def kernel(x, edge_index, y, W1, b1, W2, b2):
  src = edge_index[0].astype(jnp.int32)
  dst = edge_index[1].astype(jnp.int32)
  fill = jnp.full((EP - E,), NP - 1, jnp.int32)
  srcr = jnp.concatenate([src, fill]).reshape(NT * NB, BATCH)
  dstr = jnp.concatenate([dst, fill]).reshape(NT * NB, BATCH)
  x_p = jnp.pad(x, ((0, NP - N), (0, 0)))
  y_p = jnp.pad(y.astype(jnp.int32), (0, NP - N)).reshape(NP, 1)

  ones_b = jnp.ones((BATCH,), jnp.float32)
  zeros_rt = jnp.zeros((RT,), jnp.float32)
  zeros128 = jnp.zeros((BATCH, D_IN), jnp.float32)
  zeros40 = jnp.zeros((BATCH, D_OUT), jnp.float32)

  degp = _deg_kernel(dstr, ones_b, zeros_rt).reshape(NC, NP, 1)
  dinv, xs = _scale_call(degp, x_p)
  p1 = _agg128(xs, srcr, dstr, zeros128).reshape(NC, NP, D_IN)
  zs = _mlp_call(p1, xs, dinv, W1, b1.reshape(1, D_H), W2)
  p2 = _agg40(zs, srcr, dstr, zeros40).reshape(NC, NP, D_OUT)
  logits_p, loss_sum = _loss_call(p2, zs, dinv, b2.reshape(1, D_OUT), y_p)
  return loss_sum[0, 0] / N, logits_p[:N]


# speedup vs baseline: 8.3266x; 8.3266x over previous
import functools

import jax
import jax.numpy as jnp
from jax import lax
from jax.experimental import pallas as pl
from jax.experimental.pallas import tpu as pltpu
from jax.experimental.pallas import tpu_sc as plsc

N = 10000
E = 320000
D_IN = 128
D_H = 256
D_OUT = 40

NC = 2
NS = 16
NT = NC * NS
BATCH = 128
NB = 80
EP = NB * BATCH * NT
NP = 10240
RT = NP // NS
BN = 256
GRID = NP // BN


def _sc_mesh():
  return plsc.VectorSubcoreMesh(core_axis_name="c", subcore_axis_name="s")


@functools.partial(
    pl.kernel,
    out_type=jax.ShapeDtypeStruct((NC * NP,), jnp.float32),
    mesh=_sc_mesh(),
    scratch_types=[
        pltpu.VMEM((NB, BATCH), jnp.int32),
        pltpu.VMEM((BATCH,), jnp.float32),
        pltpu.VMEM((RT,), jnp.float32),
        pltpu.VMEM_SHARED((NP,), jnp.float32),
    ],
)
def _deg_kernel(dstr_hbm, ones_hbm, zeros_hbm, out_hbm, dst_v, ones_v, zrow_v,
                acc):
  c = lax.axis_index("c")
  s = lax.axis_index("s")
  w = s * NC + c
  pltpu.sync_copy(zeros_hbm, zrow_v)
  pltpu.sync_copy(zrow_v, acc.at[pl.ds(s * RT, RT)])
  pltpu.sync_copy(ones_hbm, ones_v)
  pltpu.sync_copy(dstr_hbm.at[pl.ds(w * NB, NB)], dst_v)
  plsc.subcore_barrier()

  def body(j, carry):
    pltpu.sync_copy(ones_v, acc.at[dst_v.at[j]], add=True)
    return carry

  lax.fori_loop(0, NB, body, 0)
  plsc.subcore_barrier()
  pltpu.sync_copy(acc.at[pl.ds(s * RT, RT)],
                  out_hbm.at[pl.ds(c * NP + s * RT, RT)])


def _make_agg_kernel(D):
  @functools.partial(
      pl.kernel,
      out_type=jax.ShapeDtypeStruct((NC * NP, D), jnp.float32),
      mesh=_sc_mesh(),
      scratch_types=[
          pltpu.VMEM((NB, BATCH), jnp.int32),
          pltpu.VMEM((NB, BATCH), jnp.int32),
          pltpu.VMEM((BATCH, D), jnp.float32),
          pltpu.VMEM_SHARED((NP, D), jnp.float32),
          pltpu.SemaphoreType.DMA,
      ],
  )
  def agg(xs_hbm, srcr_hbm, dstr_hbm, zeros_hbm, out_hbm, src_v, dst_v,
          rows_a, acc, sem_a):
    c = lax.axis_index("c")
    s = lax.axis_index("s")
    w = s * NC + c
    pltpu.sync_copy(zeros_hbm, rows_a)
    for k in range(RT // BATCH):
      pltpu.sync_copy(rows_a, acc.at[pl.ds(s * RT + k * BATCH, BATCH)])
    pltpu.sync_copy(srcr_hbm.at[pl.ds(w * NB, NB)], src_v)
    pltpu.sync_copy(dstr_hbm.at[pl.ds(w * NB, NB)], dst_v)
    plsc.subcore_barrier()

    def body(j, carry):
      pltpu.async_copy(xs_hbm.at[src_v.at[j]], rows_a, sem_a).wait()
      pltpu.sync_copy(rows_a, acc.at[dst_v.at[j]], add=True)
      return carry

    lax.fori_loop(0, NB, body, 0)
    plsc.subcore_barrier()
    pltpu.sync_copy(acc.at[pl.ds(s * RT, RT)],
                    out_hbm.at[pl.ds(c * NP + s * RT, RT)])

  return agg


_agg128 = _make_agg_kernel(D_IN)


def _scale_body(degp_ref, x_ref, dinv_ref, xs_ref):
  deg = degp_ref[0] + degp_ref[1] + 1.0
  dinv = lax.rsqrt(deg)
  dinv_ref[...] = dinv
  xs_ref[...] = x_ref[...] * dinv


def _scale_call(degp, x_p):
  return pl.pallas_call(
      _scale_body,
      grid=(GRID,),
      in_specs=[
          pl.BlockSpec((NC, BN, 1), lambda i: (0, i, 0)),
          pl.BlockSpec((BN, D_IN), lambda i: (i, 0)),
      ],
      out_specs=[
          pl.BlockSpec((BN, 1), lambda i: (i, 0)),
          pl.BlockSpec((BN, D_IN), lambda i: (i, 0)),
      ],
      out_shape=[
          jax.ShapeDtypeStruct((NP, 1), jnp.float32),
          jax.ShapeDtypeStruct((NP, D_IN), jnp.float32),
      ],
  )(degp, x_p)


def _mlp_body(p1_ref, xs_ref, dinv_ref, w1_ref, b1_ref, w2_ref, zs_ref):
  i = pl.program_id(0)
  dinv = dinv_ref[...]
  agg = (p1_ref[0] + p1_ref[1] + xs_ref[...]) * dinv
  h = jnp.maximum(
      jnp.dot(agg, w1_ref[...], preferred_element_type=jnp.float32)
      + b1_ref[...], 0.0)
  z = jnp.dot(h, w2_ref[...], preferred_element_type=jnp.float32)
  row = i * BN + lax.broadcasted_iota(jnp.int32, (BN, 1), 0)
  zs_ref[...] = jnp.where(row < N, z * dinv, 0.0)


def _mlp_call(p1, xs, dinv, W1, b1, W2p):
  return pl.pallas_call(
      _mlp_body,
      grid=(GRID,),
      in_specs=[
          pl.BlockSpec((NC, BN, D_IN), lambda i: (0, i, 0)),
          pl.BlockSpec((BN, D_IN), lambda i: (i, 0)),
          pl.BlockSpec((BN, 1), lambda i: (i, 0)),
          pl.BlockSpec((D_IN, D_H), lambda i: (0, 0)),
          pl.BlockSpec((1, D_H), lambda i: (0, 0)),
          pl.BlockSpec((D_H, D_IN), lambda i: (0, 0)),
      ],
      out_specs=pl.BlockSpec((BN, D_IN), lambda i: (i, 0)),
      out_shape=jax.ShapeDtypeStruct((NP, D_IN), jnp.float32),
  )(p1, xs, dinv, W1, b1, W2p)


def _loss_body(p2_ref, zs_ref, dinv_ref, b2_ref, y_ref, logits_ref, loss_ref):
  i = pl.program_id(0)
  full = (p2_ref[0] + p2_ref[1] + zs_ref[...]) * dinv_ref[...]
  logits = full[:, :D_OUT] + b2_ref[...]
  logits_ref[...] = logits
  m = jnp.max(logits, axis=1, keepdims=True)
  lse = jnp.log(jnp.sum(jnp.exp(logits - m), axis=1, keepdims=True)) + m
  sel = lax.broadcasted_iota(jnp.int32, (BN, D_OUT), 1) == y_ref[...]
  picked = jnp.sum(jnp.where(sel, logits, 0.0), axis=1, keepdims=True)
  row = i * BN + lax.broadcasted_iota(jnp.int32, (BN, 1), 0)
  part = jnp.sum(jnp.where(row < N, lse - picked, 0.0))

  @pl.when(i == 0)
  def _():
    loss_ref[...] = jnp.zeros((1, 1), jnp.float32)

  loss_ref[...] += part


def _loss_call(p2, zs, dinv, b2, y_p):
  return pl.pallas_call(
      _loss_body,
      grid=(GRID,),
      in_specs=[
          pl.BlockSpec((NC, BN, D_IN), lambda i: (0, i, 0)),
          pl.BlockSpec((BN, D_IN), lambda i: (i, 0)),
          pl.BlockSpec((BN, 1), lambda i: (i, 0)),
          pl.BlockSpec((1, D_OUT), lambda i: (0, 0)),
          pl.BlockSpec((BN, 1), lambda i: (i, 0)),
      ],
      out_specs=[
          pl.BlockSpec((BN, D_OUT), lambda i: (i, 0)),
          pl.BlockSpec((1, 1), lambda i: (0, 0)),
      ],
      out_shape=[
          jax.ShapeDtypeStruct((NP, D_OUT), jnp.float32),
          jax.ShapeDtypeStruct((1, 1), jnp.float32),
      ],
  )(p2, zs, dinv, b2, y_p)


def kernel(x, edge_index, y, W1, b1, W2, b2):
  src = edge_index[0].astype(jnp.int32)
  dst = edge_index[1].astype(jnp.int32)
  fill = jnp.full((EP - E,), NP - 1, jnp.int32)
  srcr = jnp.concatenate([src, fill]).reshape(NT * NB, BATCH)
  dstr = jnp.concatenate([dst, fill]).reshape(NT * NB, BATCH)
  x_p = jnp.pad(x, ((0, NP - N), (0, 0)))
  y_p = jnp.pad(y.astype(jnp.int32), (0, NP - N)).reshape(NP, 1)

  ones_b = jnp.ones((BATCH,), jnp.float32)
  zeros_rt = jnp.zeros((RT,), jnp.float32)
  zeros128 = jnp.zeros((BATCH, D_IN), jnp.float32)
  W2p = jnp.pad(W2, ((0, 0), (0, D_IN - D_OUT)))

  degp = _deg_kernel(dstr, ones_b, zeros_rt).reshape(NC, NP, 1)
  dinv, xs = _scale_call(degp, x_p)
  p1 = _agg128(xs, srcr, dstr, zeros128).reshape(NC, NP, D_IN)
  zs = _mlp_call(p1, xs, dinv, W1, b1.reshape(1, D_H), W2p)
  p2 = _agg128(zs, srcr, dstr, zeros128).reshape(NC, NP, D_IN)
  logits_p, loss_sum = _loss_call(p2, zs, dinv, b2.reshape(1, D_OUT), y_p)
  return loss_sum[0, 0] / N, logits_p[:N]


# speedup vs baseline: 8.9076x; 1.0698x over previous
import functools

import jax
import jax.numpy as jnp
from jax import lax
from jax.experimental import pallas as pl
from jax.experimental.pallas import tpu as pltpu
from jax.experimental.pallas import tpu_sc as plsc

N = 10000
E = 320000
D_IN = 128
D_H = 256
D_OUT = 40

NC = 2
NS = 16
NT = NC * NS
BATCH = 128
NB = 80
EP = NB * BATCH * NT
NP = 10240
RT = NP // NS
BN = 256
GRID = NP // BN


def _sc_mesh():
  return plsc.VectorSubcoreMesh(core_axis_name="c", subcore_axis_name="s")


@functools.partial(
    pl.kernel,
    out_type=jax.ShapeDtypeStruct((NC * NP,), jnp.float32),
    mesh=_sc_mesh(),
    scratch_types=[
        pltpu.VMEM((NB, BATCH), jnp.int32),
        pltpu.VMEM((BATCH,), jnp.float32),
        pltpu.VMEM((RT,), jnp.float32),
        pltpu.VMEM_SHARED((NP,), jnp.float32),
    ],
)
def _deg_kernel(dstr_hbm, ones_hbm, zeros_hbm, out_hbm, dst_v, ones_v, zrow_v,
                acc):
  c = lax.axis_index("c")
  s = lax.axis_index("s")
  w = s * NC + c
  pltpu.sync_copy(zeros_hbm, zrow_v)
  pltpu.sync_copy(zrow_v, acc.at[pl.ds(s * RT, RT)])
  pltpu.sync_copy(ones_hbm, ones_v)
  pltpu.sync_copy(dstr_hbm.at[pl.ds(w * NB, NB)], dst_v)
  plsc.subcore_barrier()

  def body(j, carry):
    pltpu.sync_copy(ones_v, acc.at[dst_v.at[j]], add=True)
    return carry

  lax.fori_loop(0, NB, body, 0)
  plsc.subcore_barrier()
  pltpu.sync_copy(acc.at[pl.ds(s * RT, RT)],
                  out_hbm.at[pl.ds(c * NP + s * RT, RT)])


def _make_agg_kernel(D):
  CHUNK = 8

  @functools.partial(
      pl.kernel,
      out_type=jax.ShapeDtypeStruct((NC * NP, D), jnp.float32),
      mesh=_sc_mesh(),
      scratch_types=[
          pltpu.VMEM((CHUNK, BATCH), jnp.int32),
          pltpu.VMEM((CHUNK, BATCH), jnp.int32),
          pltpu.VMEM((BATCH, D), jnp.float32),
          pltpu.VMEM((BATCH, D), jnp.float32),
          pltpu.SemaphoreType.DMA,
          pltpu.SemaphoreType.DMA,
          pltpu.VMEM_SHARED((NP, D), jnp.float32),
      ],
  )
  def agg(xs_hbm, srcr_hbm, dstr_hbm, zeros_hbm, out_hbm, src_c, dst_c,
          r0, r1, g0, g1, acc):
    rows = [r0, r1]
    gsem = [g0, g1]
    c = lax.axis_index("c")
    s = lax.axis_index("s")
    w = s * NC + c
    pltpu.sync_copy(zeros_hbm, rows[0])
    for k in range(RT // BATCH):
      pltpu.sync_copy(rows[0], acc.at[pl.ds(s * RT + k * BATCH, BATCH)])
    plsc.subcore_barrier()

    def body(i, carry):
      pltpu.sync_copy(srcr_hbm.at[pl.ds(w * NB + i * CHUNK, CHUNK)], src_c)
      pltpu.sync_copy(dstr_hbm.at[pl.ds(w * NB + i * CHUNK, CHUNK)], dst_c)
      d = [
          pltpu.async_copy(xs_hbm.at[src_c.at[0]], rows[0], gsem[0]),
          pltpu.async_copy(xs_hbm.at[src_c.at[1]], rows[1], gsem[1]),
      ]
      for b in range(CHUNK):
        p = b % 2
        d[p].wait()
        pltpu.sync_copy(rows[p], acc.at[dst_c.at[b]], add=True)
        if b + 2 < CHUNK:
          d[p] = pltpu.async_copy(xs_hbm.at[src_c.at[b + 2]], rows[p], gsem[p])
      return carry

    lax.fori_loop(0, NB // CHUNK, body, 0)
    plsc.subcore_barrier()
    pltpu.sync_copy(acc.at[pl.ds(s * RT, RT)],
                    out_hbm.at[pl.ds(c * NP + s * RT, RT)])

  return agg


_agg128 = _make_agg_kernel(D_IN)


def _scale_body(degp_ref, x_ref, dinv_ref, xs_ref):
  deg = degp_ref[0] + degp_ref[1] + 1.0
  dinv = lax.rsqrt(deg)
  dinv_ref[...] = dinv
  xs_ref[...] = x_ref[...] * dinv


def _scale_call(degp, x_p):
  return pl.pallas_call(
      _scale_body,
      grid=(GRID,),
      in_specs=[
          pl.BlockSpec((NC, BN, 1), lambda i: (0, i, 0)),
          pl.BlockSpec((BN, D_IN), lambda i: (i, 0)),
      ],
      out_specs=[
          pl.BlockSpec((BN, 1), lambda i: (i, 0)),
          pl.BlockSpec((BN, D_IN), lambda i: (i, 0)),
      ],
      out_shape=[
          jax.ShapeDtypeStruct((NP, 1), jnp.float32),
          jax.ShapeDtypeStruct((NP, D_IN), jnp.float32),
      ],
  )(degp, x_p)


def _mlp_body(p1_ref, xs_ref, dinv_ref, w1_ref, b1_ref, w2_ref, zs_ref):
  i = pl.program_id(0)
  dinv = dinv_ref[...]
  agg = (p1_ref[0] + p1_ref[1] + xs_ref[...]) * dinv
  h = jnp.maximum(
      jnp.dot(agg, w1_ref[...], preferred_element_type=jnp.float32)
      + b1_ref[...], 0.0)
  z = jnp.dot(h, w2_ref[...], preferred_element_type=jnp.float32)
  row = i * BN + lax.broadcasted_iota(jnp.int32, (BN, 1), 0)
  zs_ref[...] = jnp.where(row < N, z * dinv, 0.0)


def _mlp_call(p1, xs, dinv, W1, b1, W2p):
  return pl.pallas_call(
      _mlp_body,
      grid=(GRID,),
      in_specs=[
          pl.BlockSpec((NC, BN, D_IN), lambda i: (0, i, 0)),
          pl.BlockSpec((BN, D_IN), lambda i: (i, 0)),
          pl.BlockSpec((BN, 1), lambda i: (i, 0)),
          pl.BlockSpec((D_IN, D_H), lambda i: (0, 0)),
          pl.BlockSpec((1, D_H), lambda i: (0, 0)),
          pl.BlockSpec((D_H, D_IN), lambda i: (0, 0)),
      ],
      out_specs=pl.BlockSpec((BN, D_IN), lambda i: (i, 0)),
      out_shape=jax.ShapeDtypeStruct((NP, D_IN), jnp.float32),
  )(p1, xs, dinv, W1, b1, W2p)


def _loss_body(p2_ref, zs_ref, dinv_ref, b2_ref, y_ref, logits_ref, loss_ref):
  i = pl.program_id(0)
  full = (p2_ref[0] + p2_ref[1] + zs_ref[...]) * dinv_ref[...]
  logits = full[:, :D_OUT] + b2_ref[...]
  logits_ref[...] = logits
  m = jnp.max(logits, axis=1, keepdims=True)
  lse = jnp.log(jnp.sum(jnp.exp(logits - m), axis=1, keepdims=True)) + m
  sel = lax.broadcasted_iota(jnp.int32, (BN, D_OUT), 1) == y_ref[...]
  picked = jnp.sum(jnp.where(sel, logits, 0.0), axis=1, keepdims=True)
  row = i * BN + lax.broadcasted_iota(jnp.int32, (BN, 1), 0)
  part = jnp.sum(jnp.where(row < N, lse - picked, 0.0))

  @pl.when(i == 0)
  def _():
    loss_ref[...] = jnp.zeros((1, 1), jnp.float32)

  loss_ref[...] += part


def _loss_call(p2, zs, dinv, b2, y_p):
  return pl.pallas_call(
      _loss_body,
      grid=(GRID,),
      in_specs=[
          pl.BlockSpec((NC, BN, D_IN), lambda i: (0, i, 0)),
          pl.BlockSpec((BN, D_IN), lambda i: (i, 0)),
          pl.BlockSpec((BN, 1), lambda i: (i, 0)),
          pl.BlockSpec((1, D_OUT), lambda i: (0, 0)),
          pl.BlockSpec((BN, 1), lambda i: (i, 0)),
      ],
      out_specs=[
          pl.BlockSpec((BN, D_OUT), lambda i: (i, 0)),
          pl.BlockSpec((1, 1), lambda i: (0, 0)),
      ],
      out_shape=[
          jax.ShapeDtypeStruct((NP, D_OUT), jnp.float32),
          jax.ShapeDtypeStruct((1, 1), jnp.float32),
      ],
  )(p2, zs, dinv, b2, y_p)


def kernel(x, edge_index, y, W1, b1, W2, b2):
  src = edge_index[0].astype(jnp.int32)
  dst = edge_index[1].astype(jnp.int32)
  fill = jnp.full((EP - E,), NP - 1, jnp.int32)
  srcr = jnp.concatenate([src, fill]).reshape(NT * NB, BATCH)
  dstr = jnp.concatenate([dst, fill]).reshape(NT * NB, BATCH)
  x_p = jnp.pad(x, ((0, NP - N), (0, 0)))
  y_p = jnp.pad(y.astype(jnp.int32), (0, NP - N)).reshape(NP, 1)

  ones_b = jnp.ones((BATCH,), jnp.float32)
  zeros_rt = jnp.zeros((RT,), jnp.float32)
  zeros128 = jnp.zeros((BATCH, D_IN), jnp.float32)
  W2p = jnp.pad(W2, ((0, 0), (0, D_IN - D_OUT)))

  degp = _deg_kernel(dstr, ones_b, zeros_rt).reshape(NC, NP, 1)
  dinv, xs = _scale_call(degp, x_p)
  p1 = _agg128(xs, srcr, dstr, zeros128).reshape(NC, NP, D_IN)
  zs = _mlp_call(p1, xs, dinv, W1, b1.reshape(1, D_H), W2p)
  p2 = _agg128(zs, srcr, dstr, zeros128).reshape(NC, NP, D_IN)
  logits_p, loss_sum = _loss_call(p2, zs, dinv, b2.reshape(1, D_OUT), y_p)
  return loss_sum[0, 0] / N, logits_p[:N]


# speedup vs baseline: 9.8378x; 1.1044x over previous
import functools

import jax
import jax.numpy as jnp
from jax import lax
from jax.experimental import pallas as pl
from jax.experimental.pallas import tpu as pltpu
from jax.experimental.pallas import tpu_sc as plsc

N = 10000
E = 320000
D_IN = 128
D_H = 256
D_OUT = 40

NC = 2
NS = 16
NT = NC * NS
BATCH = 128
NB = 80
NB0 = 128
NB1 = 2 * NB - NB0
EP = NB * BATCH * NT
NP = 10240
RT = NP // NS
BN = 256
GRID = NP // BN


def _sc_mesh():
  return plsc.VectorSubcoreMesh(core_axis_name="c", subcore_axis_name="s")


@functools.partial(
    pl.kernel,
    out_type=jax.ShapeDtypeStruct((NC * NP,), jnp.float32),
    mesh=_sc_mesh(),
    scratch_types=[
        pltpu.VMEM((NB, BATCH), jnp.int32),
        pltpu.VMEM((BATCH,), jnp.float32),
        pltpu.VMEM((RT,), jnp.float32),
        pltpu.VMEM_SHARED((NP,), jnp.float32),
    ],
)
def _deg_kernel(dstr_hbm, ones_hbm, zeros_hbm, out_hbm, dst_v, ones_v, zrow_v,
                acc):
  c = lax.axis_index("c")
  s = lax.axis_index("s")
  w = s * NC + c
  pltpu.sync_copy(zeros_hbm, zrow_v)
  pltpu.sync_copy(zrow_v, acc.at[pl.ds(s * RT, RT)])
  pltpu.sync_copy(ones_hbm, ones_v)
  pltpu.sync_copy(dstr_hbm.at[pl.ds(w * NB, NB)], dst_v)
  plsc.subcore_barrier()

  def body(j, carry):
    pltpu.sync_copy(ones_v, acc.at[dst_v.at[j]], add=True)
    return carry

  lax.fori_loop(0, NB, body, 0)
  plsc.subcore_barrier()
  pltpu.sync_copy(acc.at[pl.ds(s * RT, RT)],
                  out_hbm.at[pl.ds(c * NP + s * RT, RT)])


def _make_agg_kernel(D):
  CHUNK = 8

  @functools.partial(
      pl.kernel,
      out_type=jax.ShapeDtypeStruct((NC * NP, D), jnp.float32),
      mesh=_sc_mesh(),
      scratch_types=[
          pltpu.VMEM((CHUNK, BATCH), jnp.int32),
          pltpu.VMEM((CHUNK, BATCH), jnp.int32),
          pltpu.VMEM((BATCH, D), jnp.float32),
          pltpu.VMEM((BATCH, D), jnp.float32),
          pltpu.SemaphoreType.DMA,
          pltpu.SemaphoreType.DMA,
          pltpu.VMEM_SHARED((NP, D), jnp.float32),
      ],
  )
  def agg(xs_hbm, srcr_hbm, dstr_hbm, zeros_hbm, out_hbm, src_c, dst_c,
          r0, r1, g0, g1, acc):
    rows = [r0, r1]
    gsem = [g0, g1]
    c = lax.axis_index("c")
    s = lax.axis_index("s")
    base = jnp.where(c == 0, s * NB0, NS * NB0 + s * NB1)
    nb = jnp.where(c == 0, NB0, NB1)
    pltpu.sync_copy(zeros_hbm, rows[0])
    for k in range(RT // BATCH):
      pltpu.sync_copy(rows[0], acc.at[pl.ds(s * RT + k * BATCH, BATCH)])
    plsc.subcore_barrier()

    def body(i, carry):
      pltpu.sync_copy(srcr_hbm.at[pl.ds(base + i * CHUNK, CHUNK)], src_c)
      pltpu.sync_copy(dstr_hbm.at[pl.ds(base + i * CHUNK, CHUNK)], dst_c)
      d = [
          pltpu.async_copy(xs_hbm.at[src_c.at[0]], rows[0], gsem[0]),
          pltpu.async_copy(xs_hbm.at[src_c.at[1]], rows[1], gsem[1]),
      ]
      for b in range(CHUNK):
        p = b % 2
        d[p].wait()
        pltpu.sync_copy(rows[p], acc.at[dst_c.at[b]], add=True)
        if b + 2 < CHUNK:
          d[p] = pltpu.async_copy(xs_hbm.at[src_c.at[b + 2]], rows[p], gsem[p])
      return carry

    lax.fori_loop(0, nb // CHUNK, body, 0)
    plsc.subcore_barrier()
    pltpu.sync_copy(acc.at[pl.ds(s * RT, RT)],
                    out_hbm.at[pl.ds(c * NP + s * RT, RT)])

  return agg


_agg128 = _make_agg_kernel(D_IN)


def _scale_body(degp_ref, x_ref, dinv_ref, xs_ref):
  deg = degp_ref[0] + degp_ref[1] + 1.0
  dinv = lax.rsqrt(deg)
  dinv_ref[...] = dinv
  xs_ref[...] = x_ref[...] * dinv


def _scale_call(degp, x_p):
  return pl.pallas_call(
      _scale_body,
      grid=(GRID,),
      in_specs=[
          pl.BlockSpec((NC, BN, 1), lambda i: (0, i, 0)),
          pl.BlockSpec((BN, D_IN), lambda i: (i, 0)),
      ],
      out_specs=[
          pl.BlockSpec((BN, 1), lambda i: (i, 0)),
          pl.BlockSpec((BN, D_IN), lambda i: (i, 0)),
      ],
      out_shape=[
          jax.ShapeDtypeStruct((NP, 1), jnp.float32),
          jax.ShapeDtypeStruct((NP, D_IN), jnp.float32),
      ],
  )(degp, x_p)


def _mlp_body(p1_ref, xs_ref, dinv_ref, w1_ref, b1_ref, w2_ref, zs_ref):
  i = pl.program_id(0)
  dinv = dinv_ref[...]
  agg = (p1_ref[0] + p1_ref[1] + xs_ref[...]) * dinv
  h = jnp.maximum(
      jnp.dot(agg, w1_ref[...], preferred_element_type=jnp.float32)
      + b1_ref[...], 0.0)
  z = jnp.dot(h, w2_ref[...], preferred_element_type=jnp.float32)
  row = i * BN + lax.broadcasted_iota(jnp.int32, (BN, 1), 0)
  zs_ref[...] = jnp.where(row < N, z * dinv, 0.0)


def _mlp_call(p1, xs, dinv, W1, b1, W2p):
  return pl.pallas_call(
      _mlp_body,
      grid=(GRID,),
      in_specs=[
          pl.BlockSpec((NC, BN, D_IN), lambda i: (0, i, 0)),
          pl.BlockSpec((BN, D_IN), lambda i: (i, 0)),
          pl.BlockSpec((BN, 1), lambda i: (i, 0)),
          pl.BlockSpec((D_IN, D_H), lambda i: (0, 0)),
          pl.BlockSpec((1, D_H), lambda i: (0, 0)),
          pl.BlockSpec((D_H, D_IN), lambda i: (0, 0)),
      ],
      out_specs=pl.BlockSpec((BN, D_IN), lambda i: (i, 0)),
      out_shape=jax.ShapeDtypeStruct((NP, D_IN), jnp.float32),
  )(p1, xs, dinv, W1, b1, W2p)


def _loss_body(p2_ref, zs_ref, dinv_ref, b2_ref, y_ref, logits_ref, loss_ref):
  i = pl.program_id(0)
  full = (p2_ref[0] + p2_ref[1] + zs_ref[...]) * dinv_ref[...]
  logits = full[:, :D_OUT] + b2_ref[...]
  logits_ref[...] = logits
  m = jnp.max(logits, axis=1, keepdims=True)
  lse = jnp.log(jnp.sum(jnp.exp(logits - m), axis=1, keepdims=True)) + m
  sel = lax.broadcasted_iota(jnp.int32, (BN, D_OUT), 1) == y_ref[...]
  picked = jnp.sum(jnp.where(sel, logits, 0.0), axis=1, keepdims=True)
  row = i * BN + lax.broadcasted_iota(jnp.int32, (BN, 1), 0)
  part = jnp.sum(jnp.where(row < N, lse - picked, 0.0))

  @pl.when(i == 0)
  def _():
    loss_ref[...] = jnp.zeros((1, 1), jnp.float32)

  loss_ref[...] += part


def _loss_call(p2, zs, dinv, b2, y_p):
  return pl.pallas_call(
      _loss_body,
      grid=(GRID,),
      in_specs=[
          pl.BlockSpec((NC, BN, D_IN), lambda i: (0, i, 0)),
          pl.BlockSpec((BN, D_IN), lambda i: (i, 0)),
          pl.BlockSpec((BN, 1), lambda i: (i, 0)),
          pl.BlockSpec((1, D_OUT), lambda i: (0, 0)),
          pl.BlockSpec((BN, 1), lambda i: (i, 0)),
      ],
      out_specs=[
          pl.BlockSpec((BN, D_OUT), lambda i: (i, 0)),
          pl.BlockSpec((1, 1), lambda i: (0, 0)),
      ],
      out_shape=[
          jax.ShapeDtypeStruct((NP, D_OUT), jnp.float32),
          jax.ShapeDtypeStruct((1, 1), jnp.float32),
      ],
  )(p2, zs, dinv, b2, y_p)


def kernel(x, edge_index, y, W1, b1, W2, b2):
  src = edge_index[0].astype(jnp.int32)
  dst = edge_index[1].astype(jnp.int32)
  fill = jnp.full((EP - E,), NP - 1, jnp.int32)
  srcr = jnp.concatenate([src, fill]).reshape(NT * NB, BATCH)
  dstr = jnp.concatenate([dst, fill]).reshape(NT * NB, BATCH)
  x_p = jnp.pad(x, ((0, NP - N), (0, 0)))
  y_p = jnp.pad(y.astype(jnp.int32), (0, NP - N)).reshape(NP, 1)

  ones_b = jnp.ones((BATCH,), jnp.float32)
  zeros_rt = jnp.zeros((RT,), jnp.float32)
  zeros128 = jnp.zeros((BATCH, D_IN), jnp.float32)
  W2p = jnp.pad(W2, ((0, 0), (0, D_IN - D_OUT)))

  degp = _deg_kernel(dstr, ones_b, zeros_rt).reshape(NC, NP, 1)
  dinv, xs = _scale_call(degp, x_p)
  p1 = _agg128(xs, srcr, dstr, zeros128).reshape(NC, NP, D_IN)
  zs = _mlp_call(p1, xs, dinv, W1, b1.reshape(1, D_H), W2p)
  p2 = _agg128(zs, srcr, dstr, zeros128).reshape(NC, NP, D_IN)
  logits_p, loss_sum = _loss_call(p2, zs, dinv, b2.reshape(1, D_OUT), y_p)
  return loss_sum[0, 0] / N, logits_p[:N]


# speedup vs baseline: 9.9247x; 1.0088x over previous
import functools

import jax
import jax.numpy as jnp
from jax import lax
from jax.experimental import pallas as pl
from jax.experimental.pallas import tpu as pltpu
from jax.experimental.pallas import tpu_sc as plsc

N = 10000
E = 320000
D_IN = 128
D_H = 256
D_OUT = 40

NC = 2
NS = 16
NT = NC * NS
BATCH = 128
NB = 80
NB0 = 128
NB1 = 2 * NB - NB0
EP = NB * BATCH * NT
NP = 10240
RT = NP // NS
BN = 256
GRID = NP // BN


def _sc_mesh():
  return plsc.VectorSubcoreMesh(core_axis_name="c", subcore_axis_name="s")


@functools.partial(
    pl.kernel,
    out_type=jax.ShapeDtypeStruct((NC * NP,), jnp.float32),
    mesh=_sc_mesh(),
    scratch_types=[
        pltpu.VMEM((NB, BATCH), jnp.int32),
        pltpu.VMEM((BATCH,), jnp.float32),
        pltpu.VMEM((RT,), jnp.float32),
        pltpu.VMEM_SHARED((NP,), jnp.float32),
    ],
)
def _deg_kernel(dstr_hbm, ones_hbm, zeros_hbm, out_hbm, dst_v, ones_v, zrow_v,
                acc):
  c = lax.axis_index("c")
  s = lax.axis_index("s")
  w = s * NC + c
  pltpu.sync_copy(zeros_hbm, zrow_v)
  pltpu.sync_copy(zrow_v, acc.at[pl.ds(s * RT, RT)])
  pltpu.sync_copy(ones_hbm, ones_v)
  pltpu.sync_copy(dstr_hbm.at[pl.ds(w * NB, NB)], dst_v)
  plsc.subcore_barrier()

  def body(j, carry):
    pltpu.sync_copy(ones_v, acc.at[dst_v.at[j]], add=True)
    return carry

  lax.fori_loop(0, NB, body, 0)
  plsc.subcore_barrier()
  pltpu.sync_copy(acc.at[pl.ds(s * RT, RT)],
                  out_hbm.at[pl.ds(c * NP + s * RT, RT)])


def _make_agg_kernel(D):
  CHUNK = 8

  @functools.partial(
      pl.kernel,
      out_type=jax.ShapeDtypeStruct((NC * NP, D), jnp.float32),
      mesh=_sc_mesh(),
      scratch_types=[
          pltpu.VMEM((CHUNK, BATCH), jnp.int32),
          pltpu.VMEM((CHUNK, BATCH), jnp.int32),
          pltpu.VMEM((BATCH, D), jnp.float32),
          pltpu.VMEM((BATCH, D), jnp.float32),
          pltpu.SemaphoreType.DMA,
          pltpu.SemaphoreType.DMA,
          pltpu.VMEM_SHARED((NP, D), jnp.float32),
      ],
  )
  def agg(xs_hbm, srcr_hbm, dstr_hbm, out_hbm, src_c, dst_c,
          r0, r1, g0, g1, acc):
    rows = [r0, r1]
    gsem = [g0, g1]
    c = lax.axis_index("c")
    s = lax.axis_index("s")
    base = jnp.where(c == 0, s * NB0, NS * NB0 + s * NB1)
    nb = jnp.where(c == 0, NB0, NB1)
    zv = jnp.zeros((16,), jnp.float32)

    def zbody(i, carry):
      for j in range(D // 16):
        rows[0][i, pl.ds(j * 16, 16)] = zv
      return carry

    lax.fori_loop(0, BATCH, zbody, 0)
    for k in range(RT // BATCH):
      pltpu.sync_copy(rows[0], acc.at[pl.ds(s * RT + k * BATCH, BATCH)])
    plsc.subcore_barrier()

    def body(i, carry):
      pltpu.sync_copy(srcr_hbm.at[pl.ds(base + i * CHUNK, CHUNK)], src_c)
      pltpu.sync_copy(dstr_hbm.at[pl.ds(base + i * CHUNK, CHUNK)], dst_c)
      d = [
          pltpu.async_copy(xs_hbm.at[src_c.at[0]], rows[0], gsem[0]),
          pltpu.async_copy(xs_hbm.at[src_c.at[1]], rows[1], gsem[1]),
      ]
      for b in range(CHUNK):
        p = b % 2
        d[p].wait()
        pltpu.sync_copy(rows[p], acc.at[dst_c.at[b]], add=True)
        if b + 2 < CHUNK:
          d[p] = pltpu.async_copy(xs_hbm.at[src_c.at[b + 2]], rows[p], gsem[p])
      return carry

    lax.fori_loop(0, nb // CHUNK, body, 0)
    plsc.subcore_barrier()
    dw = []
    for k in range(RT // BATCH):
      p = k % 2
      if k >= 2:
        dw[k - 2].wait()
      pltpu.sync_copy(acc.at[pl.ds(s * RT + k * BATCH, BATCH)], rows[p])
      dw.append(
          pltpu.async_copy(
              rows[p], out_hbm.at[pl.ds(c * NP + s * RT + k * BATCH, BATCH)],
              gsem[p]))
    dw[-2].wait()
    dw[-1].wait()

  return agg


_agg128 = _make_agg_kernel(D_IN)


def _scale_body(degp_ref, x_ref, dinv_ref, xs_ref):
  deg = degp_ref[0] + degp_ref[1] + 1.0
  dinv = lax.rsqrt(deg)
  dinv_ref[...] = dinv
  xs_ref[...] = x_ref[...] * dinv


def _scale_call(degp, x_p):
  return pl.pallas_call(
      _scale_body,
      grid=(GRID,),
      in_specs=[
          pl.BlockSpec((NC, BN, 1), lambda i: (0, i, 0)),
          pl.BlockSpec((BN, D_IN), lambda i: (i, 0)),
      ],
      out_specs=[
          pl.BlockSpec((BN, 1), lambda i: (i, 0)),
          pl.BlockSpec((BN, D_IN), lambda i: (i, 0)),
      ],
      out_shape=[
          jax.ShapeDtypeStruct((NP, 1), jnp.float32),
          jax.ShapeDtypeStruct((NP, D_IN), jnp.float32),
      ],
  )(degp, x_p)


def _mlp_body(p1_ref, xs_ref, dinv_ref, w1_ref, b1_ref, w2_ref, zs_ref):
  i = pl.program_id(0)
  dinv = dinv_ref[...]
  agg = (p1_ref[0] + p1_ref[1] + xs_ref[...]) * dinv
  h = jnp.maximum(
      jnp.dot(agg, w1_ref[...], preferred_element_type=jnp.float32)
      + b1_ref[...], 0.0)
  z = jnp.dot(h, w2_ref[...], preferred_element_type=jnp.float32)
  row = i * BN + lax.broadcasted_iota(jnp.int32, (BN, 1), 0)
  zs_ref[...] = jnp.where(row < N, z * dinv, 0.0)


def _mlp_call(p1, xs, dinv, W1, b1, W2p):
  return pl.pallas_call(
      _mlp_body,
      grid=(GRID,),
      in_specs=[
          pl.BlockSpec((NC, BN, D_IN), lambda i: (0, i, 0)),
          pl.BlockSpec((BN, D_IN), lambda i: (i, 0)),
          pl.BlockSpec((BN, 1), lambda i: (i, 0)),
          pl.BlockSpec((D_IN, D_H), lambda i: (0, 0)),
          pl.BlockSpec((1, D_H), lambda i: (0, 0)),
          pl.BlockSpec((D_H, D_IN), lambda i: (0, 0)),
      ],
      out_specs=pl.BlockSpec((BN, D_IN), lambda i: (i, 0)),
      out_shape=jax.ShapeDtypeStruct((NP, D_IN), jnp.float32),
  )(p1, xs, dinv, W1, b1, W2p)


def _loss_body(p2_ref, zs_ref, dinv_ref, b2_ref, y_ref, logits_ref, loss_ref):
  i = pl.program_id(0)
  full = (p2_ref[0] + p2_ref[1] + zs_ref[...]) * dinv_ref[...]
  logits = full[:, :D_OUT] + b2_ref[...]
  logits_ref[...] = logits
  m = jnp.max(logits, axis=1, keepdims=True)
  lse = jnp.log(jnp.sum(jnp.exp(logits - m), axis=1, keepdims=True)) + m
  sel = lax.broadcasted_iota(jnp.int32, (BN, D_OUT), 1) == y_ref[...]
  picked = jnp.sum(jnp.where(sel, logits, 0.0), axis=1, keepdims=True)
  row = i * BN + lax.broadcasted_iota(jnp.int32, (BN, 1), 0)
  part = jnp.sum(jnp.where(row < N, lse - picked, 0.0))

  @pl.when(i == 0)
  def _():
    loss_ref[...] = jnp.zeros((1, 1), jnp.float32)

  loss_ref[...] += part


def _loss_call(p2, zs, dinv, b2, y_p):
  return pl.pallas_call(
      _loss_body,
      grid=(GRID,),
      in_specs=[
          pl.BlockSpec((NC, BN, D_IN), lambda i: (0, i, 0)),
          pl.BlockSpec((BN, D_IN), lambda i: (i, 0)),
          pl.BlockSpec((BN, 1), lambda i: (i, 0)),
          pl.BlockSpec((1, D_OUT), lambda i: (0, 0)),
          pl.BlockSpec((BN, 1), lambda i: (i, 0)),
      ],
      out_specs=[
          pl.BlockSpec((BN, D_OUT), lambda i: (i, 0)),
          pl.BlockSpec((1, 1), lambda i: (0, 0)),
      ],
      out_shape=[
          jax.ShapeDtypeStruct((NP, D_OUT), jnp.float32),
          jax.ShapeDtypeStruct((1, 1), jnp.float32),
      ],
  )(p2, zs, dinv, b2, y_p)


def kernel(x, edge_index, y, W1, b1, W2, b2):
  src = edge_index[0].astype(jnp.int32)
  dst = edge_index[1].astype(jnp.int32)
  fill = jnp.full((EP - E,), NP - 1, jnp.int32)
  srcr = jnp.concatenate([src, fill]).reshape(NT * NB, BATCH)
  dstr = jnp.concatenate([dst, fill]).reshape(NT * NB, BATCH)
  x_p = jnp.pad(x, ((0, NP - N), (0, 0)))
  y_p = jnp.pad(y.astype(jnp.int32), (0, NP - N)).reshape(NP, 1)

  ones_b = jnp.ones((BATCH,), jnp.float32)
  zeros_rt = jnp.zeros((RT,), jnp.float32)
  W2p = jnp.pad(W2, ((0, 0), (0, D_IN - D_OUT)))

  degp = _deg_kernel(dstr, ones_b, zeros_rt).reshape(NC, NP, 1)
  dinv, xs = _scale_call(degp, x_p)
  p1 = _agg128(xs, srcr, dstr).reshape(NC, NP, D_IN)
  zs = _mlp_call(p1, xs, dinv, W1, b1.reshape(1, D_H), W2p)
  p2 = _agg128(zs, srcr, dstr).reshape(NC, NP, D_IN)
  logits_p, loss_sum = _loss_call(p2, zs, dinv, b2.reshape(1, D_OUT), y_p)
  return loss_sum[0, 0] / N, logits_p[:N]
